# baseline (speedup 1.0000x reference)
import jax
import jax.numpy as jnp
import numpy as np
from jax import lax
from jax.experimental import pallas as pl
from jax.experimental.pallas import tpu as pltpu

V = 100000
ROWS = 128
RB = 8
RG = ROWS // RB
CHUNK = 512
NFULL = V // CHUNK
TAILW = V - NFULL * CHUNK
_UNROLL = 8

_K0 = np.uint32(0)
_K1 = np.uint32(42)
_KS2 = np.uint32(_K0 ^ _K1 ^ np.uint32(0x1BD11BDA))

_TINY = np.float32(1.1754944e-38)
_NEG_INF = np.float32(float("-inf"))
_BIG_IDX = np.int32(0x7FFFFFFF)


def _threefry2x32_bits(cnt42):
    u32 = jnp.uint32
    rot = lambda v, r: (v << u32(r)) | (v >> u32(32 - r))
    ks = (_K0, _K1, _KS2)
    rotations = ((13, 15, 26, 6), (17, 29, 16, 24))
    inject = ((1, 2), (2, 0), (0, 1), (1, 2), (2, 0))

    x0 = cnt42
    x1 = cnt42
    first = True
    for g in range(5):
        for r in rotations[g % 2]:
            if first:
                first = False
            else:
                x0 = x0 + x1
            x1 = rot(x1, r)
            x1 = x1 ^ x0
        a, b = inject[g]
        x0 = x0 + ks[a]
        x1 = x1 + np.uint32(ks[b] + np.uint32(g + 1))
    return x0 ^ x1


def _gumbel_from_bits(bits):
    float_bits = (bits >> jnp.uint32(9)) | jnp.uint32(0x3F800000)
    f = lax.bitcast_convert_type(float_bits, jnp.float32) - jnp.float32(1.0)
    u = f + _TINY
    return -jnp.log(-jnp.log(u))


def _sampler_kernel(logits_ref, temps_ref, out_ref):
    g = pl.program_id(0)

    t = temps_ref[...]
    tz = t == 0.0
    safe_t = jnp.where(tz, jnp.float32(1.0), t)

    row = g * RB + lax.broadcasted_iota(jnp.int32, (RB, 1), 0)
    rowv42 = (row * V + 42).astype(jnp.uint32)

    lane_chunk = lax.broadcasted_iota(jnp.int32, (RB, CHUNK), 1)

    def sample_vals(blk, col):
        cnt42 = rowv42 + col.astype(jnp.uint32)
        gum = _gumbel_from_bits(_threefry2x32_bits(cnt42))
        return jnp.where(tz, blk, blk / safe_t + gum)

    def body(k, carry):
        accm, acci = carry
        blk = logits_ref[:, pl.ds(k * CHUNK, CHUNK)]
        col = k * CHUNK + lane_chunk
        val = sample_vals(blk, col)
        upd = val > accm
        return jnp.maximum(accm, val), jnp.where(upd, col, acci)

    accm0 = jnp.full((RB, CHUNK), _NEG_INF, jnp.float32)
    acci0 = jnp.zeros((RB, CHUNK), jnp.int32)
    accm, acci = lax.fori_loop(0, NFULL, body, (accm0, acci0),
                               unroll=_UNROLL)

    fm = jnp.max(accm, axis=1, keepdims=True)
    fi = jnp.min(jnp.where(accm == fm, acci, _BIG_IDX),
                 axis=1, keepdims=True)

    tblk = logits_ref[:, NFULL * CHUNK:V]
    tcol = NFULL * CHUNK + lax.broadcasted_iota(jnp.int32, (RB, TAILW), 1)
    tval = sample_vals(tblk, tcol)
    tm = jnp.max(tval, axis=1, keepdims=True)
    ti = jnp.min(jnp.where(tval == tm, tcol, _BIG_IDX),
                 axis=1, keepdims=True)

    out_ref[...] = jnp.where(tm > fm, ti, fi)


def kernel(logits, temperatures):
    logits = logits.astype(jnp.float32)
    temps2d = temperatures.reshape(ROWS, 1)
    out = pl.pallas_call(
        _sampler_kernel,
        grid=(RG,),
        in_specs=[
            pl.BlockSpec((RB, V), lambda g: (g, 0)),
            pl.BlockSpec((RB, 1), lambda g: (g, 0)),
        ],
        out_specs=pl.BlockSpec((RB, 1), lambda g: (g, 0)),
        out_shape=jax.ShapeDtypeStruct((ROWS, 1), jnp.int32),
        compiler_params=pltpu.CompilerParams(
            dimension_semantics=("parallel",),
        ),
    )(logits, temps2d)
    return out.reshape(ROWS)


# speedup vs baseline: 1.0497x; 1.0497x over previous
import jax
import jax.numpy as jnp
import numpy as np
from jax import lax
from jax.experimental import pallas as pl
from jax.experimental.pallas import tpu as pltpu

V = 100000
ROWS = 128
RB = 8
RG = ROWS // RB
CHUNK = 1024
NFULL = V // CHUNK
TAILW = V - NFULL * CHUNK
_UNROLL = 12

_K0 = np.uint32(0)
_K1 = np.uint32(42)
_KS2 = np.uint32(_K0 ^ _K1 ^ np.uint32(0x1BD11BDA))

_TINY = np.float32(1.1754944e-38)
_NEG_INF = np.float32(float("-inf"))
_BIG_IDX = np.int32(0x7FFFFFFF)


def _threefry2x32_bits(cnt42):
    u32 = jnp.uint32
    rot = lambda v, r: (v << u32(r)) | (v >> u32(32 - r))
    ks = (_K0, _K1, _KS2)
    rotations = ((13, 15, 26, 6), (17, 29, 16, 24))
    inject = ((1, 2), (2, 0), (0, 1), (1, 2), (2, 0))

    x0 = cnt42
    x1 = cnt42
    first = True
    for g in range(5):
        for r in rotations[g % 2]:
            if first:
                first = False
            else:
                x0 = x0 + x1
            x1 = rot(x1, r)
            x1 = x1 ^ x0
        a, b = inject[g]
        x0 = x0 + ks[a]
        x1 = x1 + np.uint32(ks[b] + np.uint32(g + 1))
    return x0 ^ x1


def _gumbel_from_bits(bits):
    float_bits = (bits >> jnp.uint32(9)) | jnp.uint32(0x3F800000)
    f = lax.bitcast_convert_type(float_bits, jnp.float32) - jnp.float32(1.0)
    u = f + _TINY
    return -jnp.log(-jnp.log(u))


def _sampler_kernel(logits_ref, temps_ref, out_ref):
    g = pl.program_id(0)

    t = temps_ref[...]
    tz = t == 0.0
    safe_t = jnp.where(tz, jnp.float32(1.0), t)

    row = g * RB + lax.broadcasted_iota(jnp.int32, (RB, 1), 0)
    rowv42 = (row * V + 42).astype(jnp.uint32)

    lane_chunk = lax.broadcasted_iota(jnp.int32, (RB, CHUNK), 1)

    def sample_vals(blk, col):
        cnt42 = rowv42 + col.astype(jnp.uint32)
        gum = _gumbel_from_bits(_threefry2x32_bits(cnt42))
        return jnp.where(tz, blk, blk / safe_t + gum)

    def body(k, carry):
        accm, acci = carry
        blk = logits_ref[:, pl.ds(k * CHUNK, CHUNK)]
        col = k * CHUNK + lane_chunk
        val = sample_vals(blk, col)
        upd = val > accm
        return jnp.maximum(accm, val), jnp.where(upd, col, acci)

    accm0 = jnp.full((RB, CHUNK), _NEG_INF, jnp.float32)
    acci0 = jnp.zeros((RB, CHUNK), jnp.int32)
    accm, acci = lax.fori_loop(0, NFULL, body, (accm0, acci0),
                               unroll=_UNROLL)

    fm = jnp.max(accm, axis=1, keepdims=True)
    fi = jnp.min(jnp.where(accm == fm, acci, _BIG_IDX),
                 axis=1, keepdims=True)

    tblk = logits_ref[:, NFULL * CHUNK:V]
    tcol = NFULL * CHUNK + lax.broadcasted_iota(jnp.int32, (RB, TAILW), 1)
    tval = sample_vals(tblk, tcol)
    tm = jnp.max(tval, axis=1, keepdims=True)
    ti = jnp.min(jnp.where(tval == tm, tcol, _BIG_IDX),
                 axis=1, keepdims=True)

    out_ref[...] = jnp.where(tm > fm, ti, fi)


def kernel(logits, temperatures):
    logits = logits.astype(jnp.float32)
    temps2d = temperatures.reshape(ROWS, 1)
    out = pl.pallas_call(
        _sampler_kernel,
        grid=(RG,),
        in_specs=[
            pl.BlockSpec((RB, V), lambda g: (g, 0)),
            pl.BlockSpec((RB, 1), lambda g: (g, 0)),
        ],
        out_specs=pl.BlockSpec((RB, 1), lambda g: (g, 0)),
        out_shape=jax.ShapeDtypeStruct((ROWS, 1), jnp.int32),
        compiler_params=pltpu.CompilerParams(
            dimension_semantics=("parallel",),
        ),
    )(logits, temps2d)
    return out.reshape(ROWS)
